# baseline (speedup 1.0000x reference)
import functools

import jax
import jax.numpy as jnp
from jax import lax
from jax.experimental import pallas as pl
from jax.experimental.pallas import tpu as pltpu
from jax.experimental.pallas import tpu_sc as plsc

D = 128
NUM_OFFSETS = 5


_ROWS_PER_BLOCK = 5000


def _table_body(cur_ref, prev_ref, nxt_ref, w1_ref, b1_ref, w2_ref, b2_ref, out_ref):
    i = pl.program_id(0)
    nb = pl.num_programs(0)
    r = _ROWS_PER_BLOCK
    cur = cur_ref[...]
    head = jnp.where(i == 0, jnp.broadcast_to(cur[:1], (2, D)), prev_ref[6:8, :])
    tail = jnp.where(
        i == nb - 1, jnp.broadcast_to(cur[r - 1 :], (2, D)), nxt_ref[0:2, :]
    )
    shifted = [
        jnp.concatenate([head, cur[: r - 2]], axis=0),
        jnp.concatenate([head[1:], cur[: r - 1]], axis=0),
        cur,
        jnp.concatenate([cur[1:], tail[:1]], axis=0),
        jnp.concatenate([cur[2:], tail], axis=0),
    ]
    acc = jnp.dot(shifted[0], w1_ref[0], preferred_element_type=jnp.float32)
    for j in range(1, NUM_OFFSETS):
        acc = acc + jnp.dot(shifted[j], w1_ref[j], preferred_element_type=jnp.float32)
    h = jnp.maximum(acc + b1_ref[...], 0.0)
    out_ref[...] = (
        jnp.dot(h, w2_ref[...], preferred_element_type=jnp.float32) + b2_ref[...]
    )


def _build_table(x, w1r, b1, w2, b2, n_rows):
    grid = n_rows // _ROWS_PER_BLOCK
    bpb = _ROWS_PER_BLOCK // 8
    n8 = n_rows // 8
    return pl.pallas_call(
        _table_body,
        grid=(grid,),
        in_specs=[
            pl.BlockSpec((_ROWS_PER_BLOCK, D), lambda i: (i, 0)),
            pl.BlockSpec((8, D), lambda i: (jnp.maximum(bpb * i - 1, 0), 0)),
            pl.BlockSpec((8, D), lambda i: (jnp.minimum(bpb * (i + 1), n8 - 1), 0)),
            pl.BlockSpec((NUM_OFFSETS, D, D), lambda i: (0, 0, 0)),
            pl.BlockSpec((1, D), lambda i: (0, 0)),
            pl.BlockSpec((D, D), lambda i: (0, 0)),
            pl.BlockSpec((1, D), lambda i: (0, 0)),
        ],
        out_specs=pl.BlockSpec((_ROWS_PER_BLOCK, D), lambda i: (i, 0)),
        out_shape=jax.ShapeDtypeStruct((n_rows, D), jnp.float32),
    )(x, x, x, w1r, b1, w2, b2)


_ROWS_PER_CHUNK = 128
_NBUF = 4


def _make_sc_gather(n_table, batch):
    info = plsc.get_sparse_core_info()
    nc, ns = info.num_cores, info.num_subcores
    nw = nc * ns
    chunks_per_worker = batch // (nw * _ROWS_PER_CHUNK)
    mesh = plsc.VectorSubcoreMesh(core_axis_name="c", subcore_axis_name="s")

    @functools.partial(
        pl.kernel,
        mesh=mesh,
        out_type=jax.ShapeDtypeStruct((batch, D), jnp.float32),
        scratch_types=(
            [pltpu.VMEM((chunks_per_worker, _ROWS_PER_CHUNK), jnp.int32)]
            + [pltpu.VMEM((_ROWS_PER_CHUNK, D), jnp.float32) for _ in range(4)]
            + [pltpu.SemaphoreType.DMA for _ in range(4)]
        ),
    )
    def gather(table_hbm, idx_hbm, out_hbm, idx_v, b0, b1, b2, b3, gsa, gsb, wsa, wsb):
        set_a, set_b = (b0, b1), (b2, b3)
        wid = lax.axis_index("s") * nc + lax.axis_index("c")
        idx_row0 = wid * chunks_per_worker
        pltpu.sync_copy(idx_hbm.at[pl.ds(idx_row0, chunks_per_worker)], idx_v)
        out_row0 = idx_row0 * _ROWS_PER_CHUNK
        last_chunk = chunks_per_worker - 1

        def fire_gathers(c0, bufs, sem, clamp=False):
            for b in range(2):
                c = jnp.minimum(c0 + b, last_chunk) if clamp else c0 + b
                pltpu.async_copy(table_hbm.at[idx_v.at[c]], bufs[b], sem)

        def drain(bufs, sem):
            for b in range(2):
                pltpu.make_async_copy(
                    table_hbm.at[pl.ds(0, _ROWS_PER_CHUNK)], bufs[b], sem
                ).wait()

        def fire_writes(c0, bufs, sem):
            for b in range(2):
                dst = out_hbm.at[
                    pl.ds(out_row0 + (c0 + b) * _ROWS_PER_CHUNK, _ROWS_PER_CHUNK)
                ]
                pltpu.async_copy(bufs[b], dst, sem)

        fire_gathers(0, set_a, gsa)

        def body(k, carry):
            c0 = 4 * k
            fire_gathers(c0 + 2, set_b, gsb)
            drain(set_a, gsa)
            fire_writes(c0, set_a, wsa)
            drain(set_a, wsa)
            fire_gathers(c0 + 4, set_a, gsa, clamp=True)
            drain(set_b, gsb)
            fire_writes(c0 + 2, set_b, wsb)
            drain(set_b, wsb)
            return carry

        lax.fori_loop(0, chunks_per_worker // 4, body, 0)
        drain(set_a, gsa)

    return gather


def kernel(x_rows, seed_indices, W1, b1, W2, b2):
    n = x_rows.shape[0]
    batch = seed_indices.shape[0]
    w1r = W1.reshape(NUM_OFFSETS, D, D)
    table = _build_table(x_rows, w1r, b1.reshape(1, D), W2, b2.reshape(1, D), n)
    idx2d = seed_indices.astype(jnp.int32).reshape(-1, _ROWS_PER_CHUNK)
    out = _make_sc_gather(n, batch)(table, idx2d)
    return out


# speedup vs baseline: 1.2017x; 1.2017x over previous
import functools

import jax
import jax.numpy as jnp
from jax import lax
from jax.experimental import pallas as pl
from jax.experimental.pallas import tpu as pltpu
from jax.experimental.pallas import tpu_sc as plsc

D = 128
NUM_OFFSETS = 5


_ROWS_PER_BLOCK = 4000


def _table_body(cur_ref, prev_ref, nxt_ref, w1_ref, b1_ref, w2_ref, b2_ref, out_ref):
    i = pl.program_id(0)
    nb = pl.num_programs(0)
    r = _ROWS_PER_BLOCK
    cur = cur_ref[...]
    head = jnp.where(i == 0, jnp.broadcast_to(cur[:1], (2, D)), prev_ref[6:8, :])
    tail = jnp.where(
        i == nb - 1, jnp.broadcast_to(cur[r - 1 :], (2, D)), nxt_ref[0:2, :]
    )
    shifted = [
        jnp.concatenate([head, cur[: r - 2]], axis=0),
        jnp.concatenate([head[1:], cur[: r - 1]], axis=0),
        cur,
        jnp.concatenate([cur[1:], tail[:1]], axis=0),
        jnp.concatenate([cur[2:], tail], axis=0),
    ]
    xcat = jnp.concatenate(shifted, axis=1)
    acc = jnp.dot(xcat, w1_ref[...], preferred_element_type=jnp.float32)
    h = jnp.maximum(acc + b1_ref[...], 0.0)
    out_ref[...] = (
        jnp.dot(h, w2_ref[...], preferred_element_type=jnp.float32) + b2_ref[...]
    )


def _build_table(x, w1r, b1, w2, b2, n_rows):
    grid = n_rows // _ROWS_PER_BLOCK
    bpb = _ROWS_PER_BLOCK // 8
    n8 = n_rows // 8
    return pl.pallas_call(
        _table_body,
        grid=(grid,),
        in_specs=[
            pl.BlockSpec((_ROWS_PER_BLOCK, D), lambda i: (i, 0)),
            pl.BlockSpec((8, D), lambda i: (jnp.maximum(bpb * i - 1, 0), 0)),
            pl.BlockSpec((8, D), lambda i: (jnp.minimum(bpb * (i + 1), n8 - 1), 0)),
            pl.BlockSpec((NUM_OFFSETS * D, D), lambda i: (0, 0)),
            pl.BlockSpec((1, D), lambda i: (0, 0)),
            pl.BlockSpec((D, D), lambda i: (0, 0)),
            pl.BlockSpec((1, D), lambda i: (0, 0)),
        ],
        out_specs=pl.BlockSpec((_ROWS_PER_BLOCK, D), lambda i: (i, 0)),
        out_shape=jax.ShapeDtypeStruct((n_rows, D), jnp.float32),
    )(x, x, x, w1r, b1, w2, b2)


_ROWS_PER_CHUNK = 128
_NBUF = 4


def _make_sc_gather(n_table, batch):
    info = plsc.get_sparse_core_info()
    nc, ns = info.num_cores, info.num_subcores
    nw = nc * ns
    chunks_per_worker = batch // (nw * _ROWS_PER_CHUNK)
    mesh = plsc.VectorSubcoreMesh(core_axis_name="c", subcore_axis_name="s")

    @functools.partial(
        pl.kernel,
        mesh=mesh,
        out_type=jax.ShapeDtypeStruct((batch, D), jnp.float32),
        scratch_types=(
            [pltpu.VMEM((chunks_per_worker, _ROWS_PER_CHUNK), jnp.int32)]
            + [pltpu.VMEM((_ROWS_PER_CHUNK, D), jnp.float32) for _ in range(4)]
            + [pltpu.SemaphoreType.DMA for _ in range(4)]
        ),
    )
    def gather(table_hbm, idx_hbm, out_hbm, idx_v, b0, b1, b2, b3, gsa, gsb, wsa, wsb):
        set_a, set_b = (b0, b1), (b2, b3)
        wid = lax.axis_index("s") * nc + lax.axis_index("c")
        idx_row0 = wid * chunks_per_worker
        pltpu.sync_copy(idx_hbm.at[pl.ds(idx_row0, chunks_per_worker)], idx_v)
        out_row0 = idx_row0 * _ROWS_PER_CHUNK
        last_chunk = chunks_per_worker - 1

        def fire_gathers(c0, bufs, sem, clamp=False):
            for b in range(2):
                c = jnp.minimum(c0 + b, last_chunk) if clamp else c0 + b
                pltpu.async_copy(table_hbm.at[idx_v.at[c]], bufs[b], sem)

        def drain(bufs, sem):
            for b in range(2):
                pltpu.make_async_copy(
                    table_hbm.at[pl.ds(0, _ROWS_PER_CHUNK)], bufs[b], sem
                ).wait()

        def fire_writes(c0, bufs, sem):
            for b in range(2):
                dst = out_hbm.at[
                    pl.ds(out_row0 + (c0 + b) * _ROWS_PER_CHUNK, _ROWS_PER_CHUNK)
                ]
                pltpu.async_copy(bufs[b], dst, sem)

        fire_gathers(0, set_a, gsa)

        def body(k, carry):
            c0 = 4 * k
            fire_gathers(c0 + 2, set_b, gsb)
            drain(set_a, gsa)
            fire_writes(c0, set_a, wsa)
            drain(set_a, wsa)
            fire_gathers(c0 + 4, set_a, gsa, clamp=True)
            drain(set_b, gsb)
            fire_writes(c0 + 2, set_b, wsb)
            drain(set_b, wsb)
            return carry

        lax.fori_loop(0, chunks_per_worker // 4, body, 0)
        drain(set_a, gsa)

    return gather


def kernel(x_rows, seed_indices, W1, b1, W2, b2):
    n = x_rows.shape[0]
    batch = seed_indices.shape[0]
    table = _build_table(x_rows, W1, b1.reshape(1, D), W2, b2.reshape(1, D), n)
    idx2d = seed_indices.astype(jnp.int32).reshape(-1, _ROWS_PER_CHUNK)
    out = _make_sc_gather(n, batch)(table, idx2d)
    return out


# speedup vs baseline: 1.2122x; 1.0087x over previous
import functools

import jax
import jax.numpy as jnp
from jax import lax
from jax.experimental import pallas as pl
from jax.experimental.pallas import tpu as pltpu
from jax.experimental.pallas import tpu_sc as plsc

D = 128
NUM_OFFSETS = 5


_ROWS_PER_BLOCK = 4000


def _table_body(cur_ref, prev_ref, nxt_ref, w1_ref, b1_ref, w2_ref, b2_ref, out_ref):
    i = pl.program_id(0)
    nb = pl.num_programs(0)
    r = _ROWS_PER_BLOCK
    cur = cur_ref[...]
    head = jnp.where(i == 0, jnp.broadcast_to(cur[:1], (2, D)), prev_ref[6:8, :])
    tail = jnp.where(
        i == nb - 1, jnp.broadcast_to(cur[r - 1 :], (2, D)), nxt_ref[0:2, :]
    )
    shifted = [
        jnp.concatenate([head, cur[: r - 2]], axis=0),
        jnp.concatenate([head[1:], cur[: r - 1]], axis=0),
        cur,
        jnp.concatenate([cur[1:], tail[:1]], axis=0),
        jnp.concatenate([cur[2:], tail], axis=0),
    ]
    xcat = jnp.concatenate(shifted, axis=1)
    acc = jnp.dot(xcat, w1_ref[...], preferred_element_type=jnp.float32)
    h = jnp.maximum(acc + b1_ref[...], 0.0)
    out_ref[...] = (
        jnp.dot(h, w2_ref[...], preferred_element_type=jnp.float32) + b2_ref[...]
    )


def _build_table(x, w1r, b1, w2, b2, n_rows):
    grid = n_rows // _ROWS_PER_BLOCK
    bpb = _ROWS_PER_BLOCK // 8
    n8 = n_rows // 8
    return pl.pallas_call(
        _table_body,
        grid=(grid,),
        in_specs=[
            pl.BlockSpec((_ROWS_PER_BLOCK, D), lambda i: (i, 0)),
            pl.BlockSpec((8, D), lambda i: (jnp.maximum(bpb * i - 1, 0), 0)),
            pl.BlockSpec((8, D), lambda i: (jnp.minimum(bpb * (i + 1), n8 - 1), 0)),
            pl.BlockSpec((NUM_OFFSETS * D, D), lambda i: (0, 0)),
            pl.BlockSpec((1, D), lambda i: (0, 0)),
            pl.BlockSpec((D, D), lambda i: (0, 0)),
            pl.BlockSpec((1, D), lambda i: (0, 0)),
        ],
        out_specs=pl.BlockSpec((_ROWS_PER_BLOCK, D), lambda i: (i, 0)),
        out_shape=jax.ShapeDtypeStruct((n_rows, D), jnp.float32),
    )(x, x, x, w1r, b1, w2, b2)


_ROWS_PER_CHUNK = 128
_NBUF = 4


def _make_sc_gather(n_table, batch):
    info = plsc.get_sparse_core_info()
    nc, ns = info.num_cores, info.num_subcores
    nw = nc * ns
    chunks_per_worker = batch // (nw * _ROWS_PER_CHUNK)
    mesh = plsc.VectorSubcoreMesh(core_axis_name="c", subcore_axis_name="s")

    @functools.partial(
        pl.kernel,
        mesh=mesh,
        out_type=jax.ShapeDtypeStruct((batch, D), jnp.float32),
        scratch_types=(
            [pltpu.VMEM((chunks_per_worker, _ROWS_PER_CHUNK), jnp.int32)]
            + [pltpu.VMEM((_ROWS_PER_CHUNK, D), jnp.float32) for _ in range(4)]
            + [pltpu.SemaphoreType.DMA for _ in range(8)]
        ),
    )
    def gather(table_hbm, idx_hbm, out_hbm, idx_v, *rest):
        bufs = rest[:4]
        gsems = rest[4:8]
        wsems = rest[8:12]
        wid = lax.axis_index("s") * nc + lax.axis_index("c")
        idx_row0 = wid * chunks_per_worker
        pltpu.sync_copy(idx_hbm.at[pl.ds(idx_row0, chunks_per_worker)], idx_v)
        out_row0 = idx_row0 * _ROWS_PER_CHUNK

        def fire_gather(c, b):
            pltpu.async_copy(table_hbm.at[idx_v.at[c]], bufs[b], gsems[b])

        def fire_write(c, b):
            dst = out_hbm.at[pl.ds(out_row0 + c * _ROWS_PER_CHUNK, _ROWS_PER_CHUNK)]
            pltpu.async_copy(bufs[b], dst, wsems[b])

        def drain(b, sems):
            pltpu.make_async_copy(
                table_hbm.at[pl.ds(0, _ROWS_PER_CHUNK)], bufs[b], sems[b]
            ).wait()

        fire_gather(0, 0)
        fire_gather(1, 1)
        for c in (0, 1):
            fire_gather(c + 2, c + 2)
            drain(c, gsems)
            fire_write(c, c)

        def step(c, u):
            bg = (u + 2) % 4
            drain(bg, wsems)
            fire_gather(c + 2, bg)
            drain(u % 4, gsems)
            fire_write(c, u % 4)

        def body(k, carry):
            for j in range(4):
                step(4 * k + 2 + j, (2 + j) % 4)
            return carry

        lax.fori_loop(0, (chunks_per_worker - 4) // 4, body, 0)
        last = chunks_per_worker - 2
        for c in (last, last + 1):
            b = c % 4
            drain(b, gsems)
            fire_write(c, b)
        for b in range(4):
            drain(b, wsems)

    return gather


def kernel(x_rows, seed_indices, W1, b1, W2, b2):
    n = x_rows.shape[0]
    batch = seed_indices.shape[0]
    table = _build_table(x_rows, W1, b1.reshape(1, D), W2, b2.reshape(1, D), n)
    idx2d = seed_indices.astype(jnp.int32).reshape(-1, _ROWS_PER_CHUNK)
    out = _make_sc_gather(n, batch)(table, idx2d)
    return out


# speedup vs baseline: 1.2124x; 1.0001x over previous
import functools

import jax
import jax.numpy as jnp
from jax import lax
from jax.experimental import pallas as pl
from jax.experimental.pallas import tpu as pltpu
from jax.experimental.pallas import tpu_sc as plsc

D = 128
NUM_OFFSETS = 5


_ROWS_PER_BLOCK = 4000


def _table_body(cur_ref, prev_ref, nxt_ref, w1_ref, b1_ref, w2_ref, b2_ref, out_ref):
    i = pl.program_id(0)
    nb = pl.num_programs(0)
    r = _ROWS_PER_BLOCK
    cur = cur_ref[...]
    head = jnp.where(i == 0, jnp.broadcast_to(cur[:1], (2, D)), prev_ref[6:8, :])
    tail = jnp.where(
        i == nb - 1, jnp.broadcast_to(cur[r - 1 :], (2, D)), nxt_ref[0:2, :]
    )
    shifted = [
        jnp.concatenate([head, cur[: r - 2]], axis=0),
        jnp.concatenate([head[1:], cur[: r - 1]], axis=0),
        cur,
        jnp.concatenate([cur[1:], tail[:1]], axis=0),
        jnp.concatenate([cur[2:], tail], axis=0),
    ]
    xcat = jnp.concatenate(shifted, axis=1)
    acc = jnp.dot(xcat, w1_ref[...], preferred_element_type=jnp.float32)
    h = jnp.maximum(acc + b1_ref[...], 0.0)
    out_ref[...] = (
        jnp.dot(h, w2_ref[...], preferred_element_type=jnp.float32) + b2_ref[...]
    )


def _build_table(x, w1r, b1, w2, b2, n_rows):
    grid = n_rows // _ROWS_PER_BLOCK
    bpb = _ROWS_PER_BLOCK // 8
    n8 = n_rows // 8
    return pl.pallas_call(
        _table_body,
        grid=(grid,),
        in_specs=[
            pl.BlockSpec((_ROWS_PER_BLOCK, D), lambda i: (i, 0)),
            pl.BlockSpec((8, D), lambda i: (jnp.maximum(bpb * i - 1, 0), 0)),
            pl.BlockSpec((8, D), lambda i: (jnp.minimum(bpb * (i + 1), n8 - 1), 0)),
            pl.BlockSpec((NUM_OFFSETS * D, D), lambda i: (0, 0)),
            pl.BlockSpec((1, D), lambda i: (0, 0)),
            pl.BlockSpec((D, D), lambda i: (0, 0)),
            pl.BlockSpec((1, D), lambda i: (0, 0)),
        ],
        out_specs=pl.BlockSpec((_ROWS_PER_BLOCK, D), lambda i: (i, 0)),
        out_shape=jax.ShapeDtypeStruct((n_rows, D), jnp.float32),
    )(x, x, x, w1r, b1, w2, b2)


_ROWS_PER_CHUNK = 128


def _make_sc_gather(batch):
    info = plsc.get_sparse_core_info()
    nc, ns = info.num_cores, info.num_subcores
    nw = nc * ns
    chunks_per_worker = batch // (nw * _ROWS_PER_CHUNK)
    mesh = plsc.VectorSubcoreMesh(core_axis_name="c", subcore_axis_name="s")

    @functools.partial(
        pl.kernel,
        mesh=mesh,
        out_type=jax.ShapeDtypeStruct((batch, D), jnp.float32),
        scratch_types=(
            [pltpu.VMEM((chunks_per_worker, _ROWS_PER_CHUNK), jnp.int32)]
            + [pltpu.VMEM((_ROWS_PER_CHUNK, D), jnp.float32) for _ in range(4)]
            + [pltpu.SemaphoreType.DMA for _ in range(8)]
        ),
    )
    def gather(table_hbm, idx_hbm, out_hbm, idx_v, *rest):
        bufs = rest[:4]
        gsems = rest[4:8]
        wsems = rest[8:12]
        wid = lax.axis_index("s") * nc + lax.axis_index("c")
        idx_row0 = wid * chunks_per_worker
        pltpu.sync_copy(idx_hbm.at[pl.ds(idx_row0, chunks_per_worker)], idx_v)
        out_row0 = idx_row0 * _ROWS_PER_CHUNK

        def fire_gather(c, b):
            pltpu.async_copy(table_hbm.at[idx_v.at[c]], bufs[b], gsems[b])

        def fire_write(c, b):
            dst = out_hbm.at[pl.ds(out_row0 + c * _ROWS_PER_CHUNK, _ROWS_PER_CHUNK)]
            pltpu.async_copy(bufs[b], dst, wsems[b])

        def drain(b, sems):
            pltpu.make_async_copy(
                table_hbm.at[pl.ds(0, _ROWS_PER_CHUNK)], bufs[b], sems[b]
            ).wait()

        fire_gather(0, 0)
        fire_gather(1, 1)
        for c in (0, 1):
            fire_gather(c + 2, c + 2)
            drain(c, gsems)
            fire_write(c, c)

        def step(c, u):
            bg = (u + 2) % 4
            drain(bg, wsems)
            fire_gather(c + 2, bg)
            drain(u % 4, gsems)
            fire_write(c, u % 4)

        def body(k, carry):
            for j in range(4):
                step(4 * k + 2 + j, (2 + j) % 4)
            return carry

        lax.fori_loop(0, (chunks_per_worker - 4) // 4, body, 0)
        last = chunks_per_worker - 2
        for c in (last, last + 1):
            b = c % 4
            drain(b, gsems)
            fire_write(c, b)
        for b in range(4):
            drain(b, wsems)

    return gather


def kernel(x_rows, seed_indices, W1, b1, W2, b2):
    n = x_rows.shape[0]
    batch = seed_indices.shape[0]
    table = _build_table(x_rows, W1, b1.reshape(1, D), W2, b2.reshape(1, D), n)
    idx2d = seed_indices.astype(jnp.int32).reshape(-1, _ROWS_PER_CHUNK)
    out = _make_sc_gather(batch)(table, idx2d)
    return out
